# baseline (speedup 1.0000x reference)
import functools

import jax
import jax.numpy as jnp
from jax import lax
from jax.experimental import pallas as pl
from jax.experimental.pallas import tpu as pltpu
from jax.experimental.pallas import tpu_sc as plsc

NC = 2
NS = 16
LN = 16
NW = NC * NS


def _dense_fwd(p, W, att_s, att_d, blk=1024):
    Np, D = p.shape

    def body(p_ref, w_ref, s_ref, d_ref, h_ref, as_ref, ad_ref):
        h = jnp.dot(p_ref[...], w_ref[...], preferred_element_type=jnp.float32)
        h_ref[...] = h
        as_ref[...] = jnp.sum(h * s_ref[...], axis=1)[None, :]
        ad_ref[...] = jnp.sum(h * d_ref[...], axis=1)[None, :]

    return pl.pallas_call(
        body,
        grid=(Np // blk,),
        in_specs=[pl.BlockSpec((blk, D), lambda i: (i, 0)),
                  pl.BlockSpec((D, D), lambda i: (0, 0)),
                  pl.BlockSpec((1, D), lambda i: (0, 0)),
                  pl.BlockSpec((1, D), lambda i: (0, 0))],
        out_specs=[pl.BlockSpec((blk, D), lambda i: (i, 0)),
                   pl.BlockSpec((1, blk), lambda i: (0, i)),
                   pl.BlockSpec((1, blk), lambda i: (0, i))],
        out_shape=[jax.ShapeDtypeStruct((Np, D), jnp.float32),
                   jax.ShapeDtypeStruct((1, Np), jnp.float32),
                   jax.ShapeDtypeStruct((1, Np), jnp.float32)],
    )(p, W, att_s[None, :], att_d[None, :])


def _combine_fwd(num, den, bias, W, att_s, att_d, blk=1024):
    _, Np, D = num.shape

    def body(n_ref, d_ref, b_ref, w_ref, s_ref, dd_ref, h_ref, as_ref, ad_ref):
        pre = (n_ref[0] + n_ref[1]) / (d_ref[0] + d_ref[1] + 1e-16) + b_ref[...]
        pre = jnp.where(pre > 0.0, pre, 0.1 * pre)
        h = jnp.dot(pre, w_ref[...], preferred_element_type=jnp.float32)
        h_ref[...] = h
        as_ref[...] = jnp.sum(h * s_ref[...], axis=1)[None, :]
        ad_ref[...] = jnp.sum(h * dd_ref[...], axis=1)[None, :]

    return pl.pallas_call(
        body,
        grid=(Np // blk,),
        in_specs=[pl.BlockSpec((NC, blk, D), lambda i: (0, i, 0)),
                  pl.BlockSpec((NC, blk, 1), lambda i: (0, i, 0)),
                  pl.BlockSpec((1, D), lambda i: (0, 0)),
                  pl.BlockSpec((D, D), lambda i: (0, 0)),
                  pl.BlockSpec((1, D), lambda i: (0, 0)),
                  pl.BlockSpec((1, D), lambda i: (0, 0))],
        out_specs=[pl.BlockSpec((blk, D), lambda i: (i, 0)),
                   pl.BlockSpec((1, blk), lambda i: (0, i)),
                   pl.BlockSpec((1, blk), lambda i: (0, i))],
        out_shape=[jax.ShapeDtypeStruct((Np, D), jnp.float32),
                   jax.ShapeDtypeStruct((1, Np), jnp.float32),
                   jax.ShapeDtypeStruct((1, Np), jnp.float32)],
    )(num, den, bias[None, :], W, att_s[None, :], att_d[None, :])


def _final_proj(num, den, bias, S, blk=1024):
    _, Np, D = num.shape
    K = S.shape[0]

    def body(n_ref, d_ref, b_ref, s_ref, o_ref):
        pre = (n_ref[0] + n_ref[1]) / (d_ref[0] + d_ref[1] + 1e-16) + b_ref[...]
        o_ref[...] = lax.dot_general(pre, s_ref[...], (((1,), (1,)), ((), ())),
                                     preferred_element_type=jnp.float32)

    return pl.pallas_call(
        body,
        grid=(Np // blk,),
        in_specs=[pl.BlockSpec((NC, blk, D), lambda i: (0, i, 0)),
                  pl.BlockSpec((NC, blk, 1), lambda i: (0, i, 0)),
                  pl.BlockSpec((1, D), lambda i: (0, 0)),
                  pl.BlockSpec((K, D), lambda i: (0, 0))],
        out_specs=pl.BlockSpec((blk, K), lambda i: (i, 0)),
        out_shape=jax.ShapeDtypeStruct((Np, K), jnp.float32),
    )(num, den, bias[None, :], S)


def _make_sc_edge(Np, D, E_pad, chunk, t_ch):
    mesh = plsc.VectorSubcoreMesh(core_axis_name="c", subcore_axis_name="s")
    rows_per_tile = Np // NS

    @functools.partial(
        pl.kernel,
        out_type=(jax.ShapeDtypeStruct((NC, Np, D), jnp.float32),
                  jax.ShapeDtypeStruct((NC, Np), jnp.float32)),
        mesh=mesh,
        compiler_params=pltpu.CompilerParams(needs_layout_passes=False),
        scratch_types=[
            pltpu.VMEM((Np,), jnp.float32),
            pltpu.VMEM((Np,), jnp.float32),
            pltpu.VMEM((chunk,), jnp.int32),
            pltpu.VMEM((chunk,), jnp.int32),
            pltpu.VMEM((chunk, D), jnp.float32),
            pltpu.VMEM((chunk,), jnp.float32),
            pltpu.VMEM_SHARED((Np, D), jnp.float32),
            pltpu.VMEM_SHARED((Np,), jnp.float32),
            pltpu.SemaphoreType.DMA,
        ],
    )
    def sc_edge(h_hbm, as_hbm, ad_hbm, src_hbm, dst_hbm, znd_hbm, zn_hbm,
                num_out, den_out,
                as_t, ad_t, sidx, didx, rows, wbuf, num_acc, den_acc, sem):
        c = lax.axis_index("c")
        s = lax.axis_index("s")
        wid = s * NC + c

        zs = s * rows_per_tile
        pltpu.sync_copy(znd_hbm.at[pl.ds(zs, rows_per_tile)],
                        num_acc.at[pl.ds(zs, rows_per_tile)])
        pltpu.sync_copy(zn_hbm.at[pl.ds(zs, rows_per_tile)],
                        den_acc.at[pl.ds(zs, rows_per_tile)])
        pltpu.sync_copy(as_hbm, as_t)
        pltpu.sync_copy(ad_hbm, ad_t)
        plsc.subcore_barrier()

        def chunk_body(t, carry):
            base = (wid * t_ch + t) * chunk
            pltpu.sync_copy(src_hbm.at[pl.ds(base, chunk)], sidx)
            pltpu.sync_copy(dst_hbm.at[pl.ds(base, chunk)], didx)
            cp = pltpu.async_copy(h_hbm.at[sidx], rows, sem)

            for j in range(chunk // LN):
                s16 = sidx[pl.ds(j * LN, LN)]
                d16 = didx[pl.ds(j * LN, LN)]
                e = plsc.load_gather(as_t, [s16]) + plsc.load_gather(ad_t, [d16])
                e = jnp.where(e > 0.0, e, 0.2 * e)
                wbuf[pl.ds(j * LN, LN)] = jnp.exp(e)

            cp.wait()

            def rowfn(r, _):
                wr = plsc.load_gather(wbuf, [jnp.full((LN,), r, jnp.int32)])
                for kk in range(D // LN):
                    rows[r, pl.ds(kk * LN, LN)] = rows[r, pl.ds(kk * LN, LN)] * wr
                return 0

            lax.fori_loop(0, chunk, rowfn, 0)
            pltpu.sync_copy(rows, num_acc.at[didx], add=True)
            pltpu.sync_copy(wbuf, den_acc.at[didx], add=True)
            return 0

        lax.fori_loop(0, t_ch, chunk_body, 0)
        plsc.subcore_barrier()

        os_ = s * rows_per_tile
        pltpu.sync_copy(num_acc.at[pl.ds(os_, rows_per_tile)],
                        num_out.at[c, pl.ds(os_, rows_per_tile)])
        pltpu.sync_copy(den_acc.at[pl.ds(os_, rows_per_tile)],
                        den_out.at[c, pl.ds(os_, rows_per_tile)])

    return sc_edge


def kernel(x, edge_index, S, W1, a1s, a1d, b1, W2, a2s, a2d, b2, W3, a3s, a3d, b3):
    N, D = x.shape
    Np = ((N + 2047) // 2048) * 2048
    E = edge_index.shape[1]
    Et = E + N
    chunk = 128
    t_ch = -(-Et // (NW * chunk))
    t_ch = ((t_ch + 3) // 4) * 4
    E_pad = NW * chunk * t_ch

    loop = jnp.arange(N, dtype=edge_index.dtype)
    src = jnp.concatenate([edge_index[0], loop])
    dst = jnp.concatenate([edge_index[1], loop])
    src = jnp.pad(src, (0, E_pad - Et), constant_values=N)
    dst = jnp.pad(dst, (0, E_pad - Et), constant_values=N)
    xp = jnp.pad(x, ((0, Np - N), (0, 0)))
    znd = jnp.zeros((Np, D), jnp.float32)
    zn = jnp.zeros((Np,), jnp.float32)

    sc_edge = _make_sc_edge(Np, D, E_pad, chunk, t_ch)

    h, asv, adv = _dense_fwd(xp, W1, a1s, a1d)
    num, den = sc_edge(h, asv.reshape(Np), adv.reshape(Np), src, dst, znd, zn)
    h, asv, adv = _combine_fwd(num, den[:, :, None], b1, W2, a2s, a2d)
    num, den = sc_edge(h, asv.reshape(Np), adv.reshape(Np), src, dst, znd, zn)
    h, asv, adv = _combine_fwd(num, den[:, :, None], b2, W3, a3s, a3d)
    num, den = sc_edge(h, asv.reshape(Np), adv.reshape(Np), src, dst, znd, zn)
    out = _final_proj(num, den[:, :, None], b3, S)
    return out[:N]


# speedup vs baseline: 2.1212x; 2.1212x over previous
import functools

import jax
import jax.numpy as jnp
from jax import lax
from jax.experimental import pallas as pl
from jax.experimental.pallas import tpu as pltpu
from jax.experimental.pallas import tpu_sc as plsc

NC = 2
NS = 16
LN = 16
NW = NC * NS


def _dense_fwd(p, W, att_s, att_d, blk=1024):
    Np, D = p.shape

    def body(p_ref, w_ref, s_ref, d_ref, h_ref, as_ref, ad_ref):
        h = jnp.dot(p_ref[...], w_ref[...], preferred_element_type=jnp.float32)
        h_ref[...] = h
        as_ref[...] = jnp.sum(h * s_ref[...], axis=1)[None, :]
        ad_ref[...] = jnp.sum(h * d_ref[...], axis=1)[None, :]

    return pl.pallas_call(
        body,
        grid=(Np // blk,),
        in_specs=[pl.BlockSpec((blk, D), lambda i: (i, 0)),
                  pl.BlockSpec((D, D), lambda i: (0, 0)),
                  pl.BlockSpec((1, D), lambda i: (0, 0)),
                  pl.BlockSpec((1, D), lambda i: (0, 0))],
        out_specs=[pl.BlockSpec((blk, D), lambda i: (i, 0)),
                   pl.BlockSpec((1, blk), lambda i: (0, i)),
                   pl.BlockSpec((1, blk), lambda i: (0, i))],
        out_shape=[jax.ShapeDtypeStruct((Np, D), jnp.float32),
                   jax.ShapeDtypeStruct((1, Np), jnp.float32),
                   jax.ShapeDtypeStruct((1, Np), jnp.float32)],
    )(p, W, att_s[None, :], att_d[None, :])


def _combine_fwd(num, den, bias, W, att_s, att_d, blk=1024):
    _, Np, D = num.shape

    def body(n_ref, d_ref, b_ref, w_ref, s_ref, dd_ref, h_ref, as_ref, ad_ref):
        pre = (n_ref[0] + n_ref[1]) / (d_ref[0] + d_ref[1] + 1e-16) + b_ref[...]
        pre = jnp.where(pre > 0.0, pre, 0.1 * pre)
        h = jnp.dot(pre, w_ref[...], preferred_element_type=jnp.float32)
        h_ref[...] = h
        as_ref[...] = jnp.sum(h * s_ref[...], axis=1)[None, :]
        ad_ref[...] = jnp.sum(h * dd_ref[...], axis=1)[None, :]

    return pl.pallas_call(
        body,
        grid=(Np // blk,),
        in_specs=[pl.BlockSpec((NC, blk, D), lambda i: (0, i, 0)),
                  pl.BlockSpec((NC, blk, 1), lambda i: (0, i, 0)),
                  pl.BlockSpec((1, D), lambda i: (0, 0)),
                  pl.BlockSpec((D, D), lambda i: (0, 0)),
                  pl.BlockSpec((1, D), lambda i: (0, 0)),
                  pl.BlockSpec((1, D), lambda i: (0, 0))],
        out_specs=[pl.BlockSpec((blk, D), lambda i: (i, 0)),
                   pl.BlockSpec((1, blk), lambda i: (0, i)),
                   pl.BlockSpec((1, blk), lambda i: (0, i))],
        out_shape=[jax.ShapeDtypeStruct((Np, D), jnp.float32),
                   jax.ShapeDtypeStruct((1, Np), jnp.float32),
                   jax.ShapeDtypeStruct((1, Np), jnp.float32)],
    )(num, den, bias[None, :], W, att_s[None, :], att_d[None, :])


def _final_proj(num, den, bias, S, blk=1024):
    _, Np, D = num.shape
    K = S.shape[0]

    def body(n_ref, d_ref, b_ref, s_ref, o_ref):
        pre = (n_ref[0] + n_ref[1]) / (d_ref[0] + d_ref[1] + 1e-16) + b_ref[...]
        o_ref[...] = lax.dot_general(pre, s_ref[...], (((1,), (1,)), ((), ())),
                                     preferred_element_type=jnp.float32)

    return pl.pallas_call(
        body,
        grid=(Np // blk,),
        in_specs=[pl.BlockSpec((NC, blk, D), lambda i: (0, i, 0)),
                  pl.BlockSpec((NC, blk, 1), lambda i: (0, i, 0)),
                  pl.BlockSpec((1, D), lambda i: (0, 0)),
                  pl.BlockSpec((K, D), lambda i: (0, 0))],
        out_specs=pl.BlockSpec((blk, K), lambda i: (i, 0)),
        out_shape=jax.ShapeDtypeStruct((Np, K), jnp.float32),
    )(num, den, bias[None, :], S)


def _make_sc_edge(Np, D, E_pad, chunk, t_ch):
    mesh = plsc.VectorSubcoreMesh(core_axis_name="c", subcore_axis_name="s")
    rows_per_tile = Np // NS

    @functools.partial(
        pl.kernel,
        out_type=(jax.ShapeDtypeStruct((NC, Np, D), jnp.float32),
                  jax.ShapeDtypeStruct((NC, Np), jnp.float32)),
        mesh=mesh,
        compiler_params=pltpu.CompilerParams(needs_layout_passes=False),
        scratch_types=[
            pltpu.VMEM((Np,), jnp.float32),
            pltpu.VMEM((Np,), jnp.float32),
            pltpu.VMEM((chunk,), jnp.int32),
            pltpu.VMEM((chunk,), jnp.int32),
            pltpu.VMEM((chunk, D), jnp.float32),
            pltpu.VMEM((chunk,), jnp.float32),
            pltpu.VMEM_SHARED((Np, D), jnp.float32),
            pltpu.VMEM_SHARED((Np,), jnp.float32),
            pltpu.SemaphoreType.DMA,
        ],
    )
    def sc_edge(h_hbm, as_hbm, ad_hbm, src_hbm, dst_hbm, znd_hbm, zn_hbm,
                num_out, den_out,
                as_t, ad_t, sidx, didx, rows, wbuf, num_acc, den_acc, sem):
        c = lax.axis_index("c")
        s = lax.axis_index("s")
        wid = s * NC + c

        zs = s * rows_per_tile
        pltpu.sync_copy(znd_hbm.at[pl.ds(zs, rows_per_tile)],
                        num_acc.at[pl.ds(zs, rows_per_tile)])
        pltpu.sync_copy(zn_hbm.at[pl.ds(zs, rows_per_tile)],
                        den_acc.at[pl.ds(zs, rows_per_tile)])
        pltpu.sync_copy(as_hbm, as_t)
        pltpu.sync_copy(ad_hbm, ad_t)
        plsc.subcore_barrier()

        def chunk_body(t, carry):
            base = (wid * t_ch + t) * chunk
            pltpu.sync_copy(src_hbm.at[pl.ds(base, chunk)], sidx)
            pltpu.sync_copy(dst_hbm.at[pl.ds(base, chunk)], didx)
            cp = pltpu.async_copy(h_hbm.at[sidx], rows, sem)

            for j in range(chunk // LN):
                s16 = sidx[pl.ds(j * LN, LN)]
                d16 = didx[pl.ds(j * LN, LN)]
                e = plsc.load_gather(as_t, [s16]) + plsc.load_gather(ad_t, [d16])
                e = jnp.where(e > 0.0, e, 0.2 * e)
                wbuf[pl.ds(j * LN, LN)] = jnp.exp(e)

            cp.wait()

            def rowfn(r, _):
                wr = plsc.load_gather(wbuf, [jnp.full((LN,), r, jnp.int32)])
                for kk in range(D // LN):
                    rows[r, pl.ds(kk * LN, LN)] = rows[r, pl.ds(kk * LN, LN)] * wr
                return 0

            lax.fori_loop(0, chunk, rowfn, 0)
            pltpu.sync_copy(rows, num_acc.at[didx], add=True)
            pltpu.sync_copy(wbuf, den_acc.at[didx], add=True)
            return 0

        lax.fori_loop(0, t_ch, chunk_body, 0)
        plsc.subcore_barrier()

        os_ = s * rows_per_tile
        pltpu.sync_copy(num_acc.at[pl.ds(os_, rows_per_tile)],
                        num_out.at[c, pl.ds(os_, rows_per_tile)])
        pltpu.sync_copy(den_acc.at[pl.ds(os_, rows_per_tile)],
                        den_out.at[c, pl.ds(os_, rows_per_tile)])

    return sc_edge


def kernel(x, edge_index, S, W1, a1s, a1d, b1, W2, a2s, a2d, b2, W3, a3s, a3d, b3):
    N, D = x.shape
    Np = ((N + 2047) // 2048) * 2048
    E = edge_index.shape[1]
    Et = E + N
    chunk = 128
    t_ch = -(-Et // (NW * chunk))
    E_pad = NW * chunk * t_ch

    pad = N + (jnp.arange(E_pad - Et, dtype=edge_index.dtype) % (Np - N))
    loop = jnp.arange(N, dtype=edge_index.dtype)
    src = jnp.concatenate([edge_index[0], loop, pad])
    dst = jnp.concatenate([edge_index[1], loop, pad])
    xp = jnp.pad(x, ((0, Np - N), (0, 0)))
    znd = jnp.zeros((Np, D), jnp.float32)
    zn = jnp.zeros((Np,), jnp.float32)

    sc_edge = _make_sc_edge(Np, D, E_pad, chunk, t_ch)

    h, asv, adv = _dense_fwd(xp, W1, a1s, a1d)
    num, den = sc_edge(h, asv.reshape(Np), adv.reshape(Np), src, dst, znd, zn)
    h, asv, adv = _combine_fwd(num, den[:, :, None], b1, W2, a2s, a2d)
    num, den = sc_edge(h, asv.reshape(Np), adv.reshape(Np), src, dst, znd, zn)
    h, asv, adv = _combine_fwd(num, den[:, :, None], b2, W3, a3s, a3d)
    num, den = sc_edge(h, asv.reshape(Np), adv.reshape(Np), src, dst, znd, zn)
    out = _final_proj(num, den[:, :, None], b3, S)
    return out[:N]


# speedup vs baseline: 2.1781x; 1.0268x over previous
import functools

import jax
import jax.numpy as jnp
from jax import lax
from jax.experimental import pallas as pl
from jax.experimental.pallas import tpu as pltpu
from jax.experimental.pallas import tpu_sc as plsc

NC = 2
NS = 16
LN = 16
NW = NC * NS


def _dense_fwd(p, W, att_s, att_d, blk=1024):
    Np, D = p.shape

    def body(p_ref, w_ref, s_ref, d_ref, h_ref, as_ref, ad_ref):
        h = jnp.dot(p_ref[...], w_ref[...], preferred_element_type=jnp.float32)
        h_ref[...] = h
        as_ref[...] = jnp.sum(h * s_ref[...], axis=1)[None, :]
        ad_ref[...] = jnp.sum(h * d_ref[...], axis=1)[None, :]

    return pl.pallas_call(
        body,
        grid=(Np // blk,),
        in_specs=[pl.BlockSpec((blk, D), lambda i: (i, 0)),
                  pl.BlockSpec((D, D), lambda i: (0, 0)),
                  pl.BlockSpec((1, D), lambda i: (0, 0)),
                  pl.BlockSpec((1, D), lambda i: (0, 0))],
        out_specs=[pl.BlockSpec((blk, D), lambda i: (i, 0)),
                   pl.BlockSpec((1, blk), lambda i: (0, i)),
                   pl.BlockSpec((1, blk), lambda i: (0, i))],
        out_shape=[jax.ShapeDtypeStruct((Np, D), jnp.float32),
                   jax.ShapeDtypeStruct((1, Np), jnp.float32),
                   jax.ShapeDtypeStruct((1, Np), jnp.float32)],
    )(p, W, att_s[None, :], att_d[None, :])


def _combine_fwd(num, den, bias, W, att_s, att_d, blk=1024):
    _, Np, D = num.shape

    def body(n_ref, d_ref, b_ref, w_ref, s_ref, dd_ref, h_ref, as_ref, ad_ref):
        pre = (n_ref[0] + n_ref[1]) / (d_ref[0] + d_ref[1] + 1e-16) + b_ref[...]
        pre = jnp.where(pre > 0.0, pre, 0.1 * pre)
        h = jnp.dot(pre, w_ref[...], preferred_element_type=jnp.float32)
        h_ref[...] = h
        as_ref[...] = jnp.sum(h * s_ref[...], axis=1)[None, :]
        ad_ref[...] = jnp.sum(h * dd_ref[...], axis=1)[None, :]

    return pl.pallas_call(
        body,
        grid=(Np // blk,),
        in_specs=[pl.BlockSpec((NC, blk, D), lambda i: (0, i, 0)),
                  pl.BlockSpec((NC, blk, 1), lambda i: (0, i, 0)),
                  pl.BlockSpec((1, D), lambda i: (0, 0)),
                  pl.BlockSpec((D, D), lambda i: (0, 0)),
                  pl.BlockSpec((1, D), lambda i: (0, 0)),
                  pl.BlockSpec((1, D), lambda i: (0, 0))],
        out_specs=[pl.BlockSpec((blk, D), lambda i: (i, 0)),
                   pl.BlockSpec((1, blk), lambda i: (0, i)),
                   pl.BlockSpec((1, blk), lambda i: (0, i))],
        out_shape=[jax.ShapeDtypeStruct((Np, D), jnp.float32),
                   jax.ShapeDtypeStruct((1, Np), jnp.float32),
                   jax.ShapeDtypeStruct((1, Np), jnp.float32)],
    )(num, den, bias[None, :], W, att_s[None, :], att_d[None, :])


def _final_proj(num, den, bias, S, blk=1024):
    _, Np, D = num.shape
    K = S.shape[0]

    def body(n_ref, d_ref, b_ref, s_ref, o_ref):
        pre = (n_ref[0] + n_ref[1]) / (d_ref[0] + d_ref[1] + 1e-16) + b_ref[...]
        o_ref[...] = lax.dot_general(pre, s_ref[...], (((1,), (1,)), ((), ())),
                                     preferred_element_type=jnp.float32)

    return pl.pallas_call(
        body,
        grid=(Np // blk,),
        in_specs=[pl.BlockSpec((NC, blk, D), lambda i: (0, i, 0)),
                  pl.BlockSpec((NC, blk, 1), lambda i: (0, i, 0)),
                  pl.BlockSpec((1, D), lambda i: (0, 0)),
                  pl.BlockSpec((K, D), lambda i: (0, 0))],
        out_specs=pl.BlockSpec((blk, K), lambda i: (i, 0)),
        out_shape=jax.ShapeDtypeStruct((Np, K), jnp.float32),
    )(num, den, bias[None, :], S)


def _make_sc_edge(Np, D, E_pad, chunk, t_ch):
    mesh = plsc.VectorSubcoreMesh(core_axis_name="c", subcore_axis_name="s")
    rows_per_tile = Np // NS

    @functools.partial(
        pl.kernel,
        out_type=(jax.ShapeDtypeStruct((NC, Np, D), jnp.float32),
                  jax.ShapeDtypeStruct((NC, Np), jnp.float32)),
        mesh=mesh,
        compiler_params=pltpu.CompilerParams(needs_layout_passes=False),
        scratch_types=[
            pltpu.VMEM((Np,), jnp.float32),
            pltpu.VMEM((Np,), jnp.float32),
            pltpu.VMEM((chunk,), jnp.int32),
            pltpu.VMEM((chunk,), jnp.int32),
            pltpu.VMEM((chunk, D), jnp.float32),
            pltpu.VMEM((chunk,), jnp.float32),
            pltpu.VMEM_SHARED((Np, D), jnp.float32),
            pltpu.VMEM_SHARED((Np,), jnp.float32),
            pltpu.SemaphoreType.DMA,
        ],
    )
    def sc_edge(h_hbm, as_hbm, ad_hbm, src_hbm, dst_hbm, znd_hbm, zn_hbm,
                num_out, den_out,
                as_t, ad_t, sidx, didx, rows, wbuf, num_acc, den_acc, sem):
        c = lax.axis_index("c")
        s = lax.axis_index("s")
        wid = s * NC + c

        zs = s * rows_per_tile
        pltpu.sync_copy(znd_hbm.at[pl.ds(zs, rows_per_tile)],
                        num_acc.at[pl.ds(zs, rows_per_tile)])
        pltpu.sync_copy(zn_hbm.at[pl.ds(zs, rows_per_tile)],
                        den_acc.at[pl.ds(zs, rows_per_tile)])
        pltpu.sync_copy(as_hbm, as_t)
        pltpu.sync_copy(ad_hbm, ad_t)
        plsc.subcore_barrier()

        def chunk_body(t, carry):
            base = (wid * t_ch + t) * chunk
            pltpu.sync_copy(src_hbm.at[pl.ds(base, chunk)], sidx)
            pltpu.sync_copy(dst_hbm.at[pl.ds(base, chunk)], didx)
            cp = pltpu.async_copy(h_hbm.at[sidx], rows, sem)

            for j in range(chunk // LN):
                s16 = sidx[pl.ds(j * LN, LN)]
                d16 = didx[pl.ds(j * LN, LN)]
                e = plsc.load_gather(as_t, [s16]) + plsc.load_gather(ad_t, [d16])
                e = jnp.where(e > 0.0, e, 0.2 * e)
                wbuf[pl.ds(j * LN, LN)] = jnp.exp(e)

            cp.wait()

            def rowfn(r, _):
                wr = plsc.load_gather(wbuf, [jnp.full((LN,), r, jnp.int32)])
                for kk in range(D // LN):
                    rows[r, pl.ds(kk * LN, LN)] = rows[r, pl.ds(kk * LN, LN)] * wr
                return 0

            lax.fori_loop(0, chunk, rowfn, 0, unroll=8)
            pltpu.sync_copy(rows, num_acc.at[didx], add=True)
            pltpu.sync_copy(wbuf, den_acc.at[didx], add=True)
            return 0

        lax.fori_loop(0, t_ch, chunk_body, 0)
        plsc.subcore_barrier()

        os_ = s * rows_per_tile
        pltpu.sync_copy(num_acc.at[pl.ds(os_, rows_per_tile)],
                        num_out.at[c, pl.ds(os_, rows_per_tile)])
        pltpu.sync_copy(den_acc.at[pl.ds(os_, rows_per_tile)],
                        den_out.at[c, pl.ds(os_, rows_per_tile)])

    return sc_edge


def kernel(x, edge_index, S, W1, a1s, a1d, b1, W2, a2s, a2d, b2, W3, a3s, a3d, b3):
    N, D = x.shape
    Np = ((N + 2047) // 2048) * 2048
    E = edge_index.shape[1]
    Et = E + N
    chunk = 128
    t_ch = -(-Et // (NW * chunk))
    E_pad = NW * chunk * t_ch

    pad = N + (jnp.arange(E_pad - Et, dtype=edge_index.dtype) % (Np - N))
    loop = jnp.arange(N, dtype=edge_index.dtype)
    src = jnp.concatenate([edge_index[0], loop, pad])
    dst = jnp.concatenate([edge_index[1], loop, pad])
    xp = jnp.pad(x, ((0, Np - N), (0, 0)))
    znd = jnp.zeros((Np, D), jnp.float32)
    zn = jnp.zeros((Np,), jnp.float32)

    sc_edge = _make_sc_edge(Np, D, E_pad, chunk, t_ch)

    h, asv, adv = _dense_fwd(xp, W1, a1s, a1d)
    num, den = sc_edge(h, asv.reshape(Np), adv.reshape(Np), src, dst, znd, zn)
    h, asv, adv = _combine_fwd(num, den[:, :, None], b1, W2, a2s, a2d)
    num, den = sc_edge(h, asv.reshape(Np), adv.reshape(Np), src, dst, znd, zn)
    h, asv, adv = _combine_fwd(num, den[:, :, None], b2, W3, a3s, a3d)
    num, den = sc_edge(h, asv.reshape(Np), adv.reshape(Np), src, dst, znd, zn)
    out = _final_proj(num, den[:, :, None], b3, S)
    return out[:N]


# speedup vs baseline: 3.7228x; 1.7092x over previous
import functools

import jax
import jax.numpy as jnp
from jax import lax
from jax.experimental import pallas as pl
from jax.experimental.pallas import tpu as pltpu
from jax.experimental.pallas import tpu_sc as plsc

NC = 2
NS = 16
LN = 16
NW = NC * NS


def _dense_fwd(p, W, att_s, att_d, blk=1024):
    Np, D = p.shape

    def body(p_ref, w_ref, s_ref, d_ref, h_ref, as_ref, ad_ref):
        h = jnp.dot(p_ref[...], w_ref[...], preferred_element_type=jnp.float32)
        h_ref[...] = h
        as_ref[...] = jnp.sum(h * s_ref[...], axis=1)[None, :]
        ad_ref[...] = jnp.sum(h * d_ref[...], axis=1)[None, :]

    return pl.pallas_call(
        body,
        grid=(Np // blk,),
        in_specs=[pl.BlockSpec((blk, D), lambda i: (i, 0)),
                  pl.BlockSpec((D, D), lambda i: (0, 0)),
                  pl.BlockSpec((1, D), lambda i: (0, 0)),
                  pl.BlockSpec((1, D), lambda i: (0, 0))],
        out_specs=[pl.BlockSpec((blk, D), lambda i: (i, 0)),
                   pl.BlockSpec((1, blk), lambda i: (0, i)),
                   pl.BlockSpec((1, blk), lambda i: (0, i))],
        out_shape=[jax.ShapeDtypeStruct((Np, D), jnp.float32),
                   jax.ShapeDtypeStruct((1, Np), jnp.float32),
                   jax.ShapeDtypeStruct((1, Np), jnp.float32)],
    )(p, W, att_s[None, :], att_d[None, :])


def _combine_fwd(num, den, bias, W, att_s, att_d, blk=1024):
    _, Np, D = num.shape

    def body(n_ref, d_ref, b_ref, w_ref, s_ref, dd_ref, h_ref, as_ref, ad_ref):
        pre = (n_ref[0] + n_ref[1]) / (d_ref[0] + d_ref[1] + 1e-16) + b_ref[...]
        pre = jnp.where(pre > 0.0, pre, 0.1 * pre)
        h = jnp.dot(pre, w_ref[...], preferred_element_type=jnp.float32)
        h_ref[...] = h
        as_ref[...] = jnp.sum(h * s_ref[...], axis=1)[None, :]
        ad_ref[...] = jnp.sum(h * dd_ref[...], axis=1)[None, :]

    return pl.pallas_call(
        body,
        grid=(Np // blk,),
        in_specs=[pl.BlockSpec((NC, blk, D), lambda i: (0, i, 0)),
                  pl.BlockSpec((NC, blk, 1), lambda i: (0, i, 0)),
                  pl.BlockSpec((1, D), lambda i: (0, 0)),
                  pl.BlockSpec((D, D), lambda i: (0, 0)),
                  pl.BlockSpec((1, D), lambda i: (0, 0)),
                  pl.BlockSpec((1, D), lambda i: (0, 0))],
        out_specs=[pl.BlockSpec((blk, D), lambda i: (i, 0)),
                   pl.BlockSpec((1, blk), lambda i: (0, i)),
                   pl.BlockSpec((1, blk), lambda i: (0, i))],
        out_shape=[jax.ShapeDtypeStruct((Np, D), jnp.float32),
                   jax.ShapeDtypeStruct((1, Np), jnp.float32),
                   jax.ShapeDtypeStruct((1, Np), jnp.float32)],
    )(num, den, bias[None, :], W, att_s[None, :], att_d[None, :])


def _final_proj(num, den, bias, S, blk=1024):
    _, Np, D = num.shape
    K = S.shape[0]

    def body(n_ref, d_ref, b_ref, s_ref, o_ref):
        pre = (n_ref[0] + n_ref[1]) / (d_ref[0] + d_ref[1] + 1e-16) + b_ref[...]
        o_ref[...] = lax.dot_general(pre, s_ref[...], (((1,), (1,)), ((), ())),
                                     preferred_element_type=jnp.float32)

    return pl.pallas_call(
        body,
        grid=(Np // blk,),
        in_specs=[pl.BlockSpec((NC, blk, D), lambda i: (0, i, 0)),
                  pl.BlockSpec((NC, blk, 1), lambda i: (0, i, 0)),
                  pl.BlockSpec((1, D), lambda i: (0, 0)),
                  pl.BlockSpec((K, D), lambda i: (0, 0))],
        out_specs=pl.BlockSpec((blk, K), lambda i: (i, 0)),
        out_shape=jax.ShapeDtypeStruct((Np, K), jnp.float32),
    )(num, den, bias[None, :], S)


def _make_sc_edge(Np, D, E_pad, chunk, t_ch):
    mesh = plsc.VectorSubcoreMesh(core_axis_name="c", subcore_axis_name="s")
    rows_per_tile = Np // NS

    @functools.partial(
        pl.kernel,
        out_type=(jax.ShapeDtypeStruct((NC, Np, D), jnp.float32),
                  jax.ShapeDtypeStruct((NC, Np), jnp.float32)),
        mesh=mesh,
        compiler_params=pltpu.CompilerParams(needs_layout_passes=False),
        scratch_types=[
            pltpu.VMEM((Np,), jnp.float32),
            pltpu.VMEM((Np,), jnp.float32),
            pltpu.VMEM((2, chunk), jnp.int32),
            pltpu.VMEM((2, chunk), jnp.int32),
            pltpu.VMEM((2, chunk), jnp.int32),
            pltpu.VMEM((2, chunk, D), jnp.float32),
            pltpu.VMEM((2, chunk), jnp.float32),
            pltpu.VMEM_SHARED((Np, D), jnp.float32),
            pltpu.VMEM_SHARED((Np,), jnp.float32),
            pltpu.SemaphoreType.DMA,
            pltpu.SemaphoreType.DMA,
            pltpu.SemaphoreType.DMA,
            pltpu.SemaphoreType.DMA,
            pltpu.SemaphoreType.DMA,
            pltpu.SemaphoreType.DMA,
            pltpu.SemaphoreType.DMA,
            pltpu.SemaphoreType.DMA,
        ],
    )
    def sc_edge(h_hbm, as_hbm, ad_hbm, src_hbm, dst_hbm, znd_hbm, zn_hbm,
                num_out, den_out,
                as_t, ad_t, sidx, didx, dscat, rows2, wbuf, num_acc, den_acc,
                gsem0, gsem1, rsem0, rsem1, dsem0, dsem1, isem0, isem1):
        c = lax.axis_index("c")
        s = lax.axis_index("s")
        wid = s * NC + c
        gsem = (gsem0, gsem1)
        rsem = (rsem0, rsem1)
        dsem = (dsem0, dsem1)
        isem = (isem0, isem1)

        zs = s * rows_per_tile
        pltpu.sync_copy(znd_hbm.at[pl.ds(zs, rows_per_tile)],
                        num_acc.at[pl.ds(zs, rows_per_tile)])
        pltpu.sync_copy(zn_hbm.at[pl.ds(zs, rows_per_tile)],
                        den_acc.at[pl.ds(zs, rows_per_tile)])
        pltpu.sync_copy(as_hbm, as_t)
        pltpu.sync_copy(ad_hbm, ad_t)
        base0 = wid * t_ch * chunk
        pltpu.sync_copy(src_hbm.at[pl.ds(base0, chunk)], sidx.at[0])
        pltpu.sync_copy(dst_hbm.at[pl.ds(base0, chunk)], didx.at[0])
        pltpu.async_copy(src_hbm.at[pl.ds(base0 + chunk, chunk)], sidx.at[1],
                         isem[1])
        pltpu.async_copy(dst_hbm.at[pl.ds(base0 + chunk, chunk)], didx.at[1],
                         isem[1])
        pltpu.async_copy(h_hbm.at[sidx.at[0]], rows2.at[0], gsem[0])
        plsc.subcore_barrier()

        def process(t, b):
            @pl.when(t >= 1)
            def _():
                pltpu.make_async_copy(znd_hbm.at[pl.ds(0, chunk)],
                                      rows2.at[1 - b], rsem[1 - b]).wait()
                pltpu.make_async_copy(zn_hbm.at[pl.ds(0, chunk)],
                                      wbuf.at[1 - b], dsem[1 - b]).wait()

            @pl.when(t + 1 < t_ch)
            def _():
                pltpu.make_async_copy(src_hbm.at[pl.ds(0, chunk)],
                                      sidx.at[1 - b], isem[1 - b]).wait()
                pltpu.make_async_copy(src_hbm.at[pl.ds(0, chunk)],
                                      didx.at[1 - b], isem[1 - b]).wait()
                pltpu.async_copy(h_hbm.at[sidx.at[1 - b]], rows2.at[1 - b],
                                 gsem[1 - b])

            for j in range(chunk // LN):
                s16 = sidx[b, pl.ds(j * LN, LN)]
                d16 = didx[b, pl.ds(j * LN, LN)]
                dscat[b, pl.ds(j * LN, LN)] = d16
                e = plsc.load_gather(as_t, [s16]) + plsc.load_gather(ad_t, [d16])
                e = jnp.where(e > 0.0, e, 0.2 * e)
                wbuf[b, pl.ds(j * LN, LN)] = jnp.exp(e)

            pltpu.async_copy(wbuf.at[b], den_acc.at[dscat.at[b]], dsem[b],
                             add=True)

            pltpu.make_async_copy(znd_hbm.at[pl.ds(0, chunk)],
                                  rows2.at[b], gsem[b]).wait()

            def rowfn(r, _):
                wr = plsc.load_gather(wbuf.at[b], [jnp.full((LN,), r, jnp.int32)])
                for kk in range(D // LN):
                    rows2[b, r, pl.ds(kk * LN, LN)] = (
                        rows2[b, r, pl.ds(kk * LN, LN)] * wr)
                return 0

            lax.fori_loop(0, chunk, rowfn, 0, unroll=8)
            pltpu.async_copy(rows2.at[b], num_acc.at[dscat.at[b]], rsem[b],
                             add=True)

            @pl.when(t + 2 < t_ch)
            def _():
                base = (wid * t_ch + t + 2) * chunk
                pltpu.async_copy(src_hbm.at[pl.ds(base, chunk)], sidx.at[b],
                                 isem[b])
                pltpu.async_copy(dst_hbm.at[pl.ds(base, chunk)], didx.at[b],
                                 isem[b])

        def pair(t2, _):
            process(t2 * 2, 0)
            process(t2 * 2 + 1, 1)
            return 0

        lax.fori_loop(0, t_ch // 2, pair, 0)

        pltpu.make_async_copy(znd_hbm.at[pl.ds(0, chunk)],
                              rows2.at[1], rsem[1]).wait()
        pltpu.make_async_copy(zn_hbm.at[pl.ds(0, chunk)],
                              wbuf.at[1], dsem[1]).wait()
        plsc.subcore_barrier()

        os_ = s * rows_per_tile
        pltpu.sync_copy(num_acc.at[pl.ds(os_, rows_per_tile)],
                        num_out.at[c, pl.ds(os_, rows_per_tile)])
        pltpu.sync_copy(den_acc.at[pl.ds(os_, rows_per_tile)],
                        den_out.at[c, pl.ds(os_, rows_per_tile)])

    return sc_edge


def kernel(x, edge_index, S, W1, a1s, a1d, b1, W2, a2s, a2d, b2, W3, a3s, a3d, b3):
    N, D = x.shape
    Np = ((N + 2047) // 2048) * 2048
    E = edge_index.shape[1]
    Et = E + N
    chunk = 80
    t_ch = -(-Et // (NW * chunk))
    t_ch += t_ch % 2
    E_pad = NW * chunk * t_ch

    pad = N + (jnp.arange(E_pad - Et, dtype=edge_index.dtype) % (Np - N))
    loop = jnp.arange(N, dtype=edge_index.dtype)
    src = jnp.concatenate([edge_index[0], loop, pad])
    dst = jnp.concatenate([edge_index[1], loop, pad])
    xp = jnp.pad(x, ((0, Np - N), (0, 0)))
    znd = jnp.zeros((Np, D), jnp.float32)
    zn = jnp.zeros((Np,), jnp.float32)

    sc_edge = _make_sc_edge(Np, D, E_pad, chunk, t_ch)

    h, asv, adv = _dense_fwd(xp, W1, a1s, a1d)
    num, den = sc_edge(h, asv.reshape(Np), adv.reshape(Np), src, dst, znd, zn)
    h, asv, adv = _combine_fwd(num, den[:, :, None], b1, W2, a2s, a2d)
    num, den = sc_edge(h, asv.reshape(Np), adv.reshape(Np), src, dst, znd, zn)
    h, asv, adv = _combine_fwd(num, den[:, :, None], b2, W3, a3s, a3d)
    num, den = sc_edge(h, asv.reshape(Np), adv.reshape(Np), src, dst, znd, zn)
    out = _final_proj(num, den[:, :, None], b3, S)
    return out[:N]


# speedup vs baseline: 3.8084x; 1.0230x over previous
import functools

import jax
import jax.numpy as jnp
from jax import lax
from jax.experimental import pallas as pl
from jax.experimental.pallas import tpu as pltpu
from jax.experimental.pallas import tpu_sc as plsc

NC = 2
NS = 16
LN = 16
NW = NC * NS


def _dense_fwd(p, W, att_s, att_d, blk=1024):
    Np, D = p.shape

    def body(p_ref, w_ref, s_ref, d_ref, h_ref, as_ref, ad_ref):
        h = jnp.dot(p_ref[...], w_ref[...], preferred_element_type=jnp.float32)
        h_ref[...] = h
        as_ref[...] = jnp.sum(h * s_ref[...], axis=1)[None, :]
        ad_ref[...] = jnp.sum(h * d_ref[...], axis=1)[None, :]

    return pl.pallas_call(
        body,
        grid=(Np // blk,),
        in_specs=[pl.BlockSpec((blk, D), lambda i: (i, 0)),
                  pl.BlockSpec((D, D), lambda i: (0, 0)),
                  pl.BlockSpec((1, D), lambda i: (0, 0)),
                  pl.BlockSpec((1, D), lambda i: (0, 0))],
        out_specs=[pl.BlockSpec((blk, D), lambda i: (i, 0)),
                   pl.BlockSpec((1, blk), lambda i: (0, i)),
                   pl.BlockSpec((1, blk), lambda i: (0, i))],
        out_shape=[jax.ShapeDtypeStruct((Np, D), jnp.float32),
                   jax.ShapeDtypeStruct((1, Np), jnp.float32),
                   jax.ShapeDtypeStruct((1, Np), jnp.float32)],
    )(p, W, att_s[None, :], att_d[None, :])


def _combine_fwd(num, den, bias, W, att_s, att_d, blk=1024):
    _, Np, D = num.shape

    def body(n_ref, d_ref, b_ref, w_ref, s_ref, dd_ref, h_ref, as_ref, ad_ref):
        pre = (n_ref[0] + n_ref[1]) / (d_ref[0] + d_ref[1] + 1e-16) + b_ref[...]
        pre = jnp.where(pre > 0.0, pre, 0.1 * pre)
        h = jnp.dot(pre, w_ref[...], preferred_element_type=jnp.float32)
        h_ref[...] = h
        as_ref[...] = jnp.sum(h * s_ref[...], axis=1)[None, :]
        ad_ref[...] = jnp.sum(h * dd_ref[...], axis=1)[None, :]

    return pl.pallas_call(
        body,
        grid=(Np // blk,),
        in_specs=[pl.BlockSpec((NC, blk, D), lambda i: (0, i, 0)),
                  pl.BlockSpec((NC, blk, 1), lambda i: (0, i, 0)),
                  pl.BlockSpec((1, D), lambda i: (0, 0)),
                  pl.BlockSpec((D, D), lambda i: (0, 0)),
                  pl.BlockSpec((1, D), lambda i: (0, 0)),
                  pl.BlockSpec((1, D), lambda i: (0, 0))],
        out_specs=[pl.BlockSpec((blk, D), lambda i: (i, 0)),
                   pl.BlockSpec((1, blk), lambda i: (0, i)),
                   pl.BlockSpec((1, blk), lambda i: (0, i))],
        out_shape=[jax.ShapeDtypeStruct((Np, D), jnp.float32),
                   jax.ShapeDtypeStruct((1, Np), jnp.float32),
                   jax.ShapeDtypeStruct((1, Np), jnp.float32)],
    )(num, den, bias[None, :], W, att_s[None, :], att_d[None, :])


def _final_proj(num, den, bias, S, blk=1024):
    _, Np, D = num.shape
    K = S.shape[0]

    def body(n_ref, d_ref, b_ref, s_ref, o_ref):
        pre = (n_ref[0] + n_ref[1]) / (d_ref[0] + d_ref[1] + 1e-16) + b_ref[...]
        o_ref[...] = lax.dot_general(pre, s_ref[...], (((1,), (1,)), ((), ())),
                                     preferred_element_type=jnp.float32)

    return pl.pallas_call(
        body,
        grid=(Np // blk,),
        in_specs=[pl.BlockSpec((NC, blk, D), lambda i: (0, i, 0)),
                  pl.BlockSpec((NC, blk, 1), lambda i: (0, i, 0)),
                  pl.BlockSpec((1, D), lambda i: (0, 0)),
                  pl.BlockSpec((K, D), lambda i: (0, 0))],
        out_specs=pl.BlockSpec((blk, K), lambda i: (i, 0)),
        out_shape=jax.ShapeDtypeStruct((Np, K), jnp.float32),
    )(num, den, bias[None, :], S)


def _make_sc_edge(Np, D, E_pad, chunk, t_ch):
    mesh = plsc.VectorSubcoreMesh(core_axis_name="c", subcore_axis_name="s")
    rows_per_tile = Np // NS

    @functools.partial(
        pl.kernel,
        out_type=(jax.ShapeDtypeStruct((NC, Np, D), jnp.float32),
                  jax.ShapeDtypeStruct((NC, Np), jnp.float32)),
        mesh=mesh,
        compiler_params=pltpu.CompilerParams(needs_layout_passes=False),
        scratch_types=[
            pltpu.VMEM((Np,), jnp.float32),
            pltpu.VMEM((Np,), jnp.float32),
            pltpu.VMEM((2, chunk), jnp.int32),
            pltpu.VMEM((2, chunk), jnp.int32),
            pltpu.VMEM((2, chunk), jnp.int32),
            pltpu.VMEM((2, chunk, D), jnp.float32),
            pltpu.VMEM((2, chunk), jnp.float32),
            pltpu.VMEM_SHARED((Np, D), jnp.float32),
            pltpu.VMEM_SHARED((Np,), jnp.float32),
            pltpu.SemaphoreType.DMA,
            pltpu.SemaphoreType.DMA,
            pltpu.SemaphoreType.DMA,
            pltpu.SemaphoreType.DMA,
            pltpu.SemaphoreType.DMA,
            pltpu.SemaphoreType.DMA,
            pltpu.SemaphoreType.DMA,
            pltpu.SemaphoreType.DMA,
        ],
    )
    def sc_edge(h_hbm, as_hbm, ad_hbm, src_hbm, dst_hbm, znd_hbm, zn_hbm,
                num_out, den_out,
                as_t, ad_t, sidx, didx, dscat, rows2, wbuf, num_acc, den_acc,
                gsem0, gsem1, rsem0, rsem1, dsem0, dsem1, isem0, isem1):
        c = lax.axis_index("c")
        s = lax.axis_index("s")
        wid = s * NC + c
        gsem = (gsem0, gsem1)
        rsem = (rsem0, rsem1)
        dsem = (dsem0, dsem1)
        isem = (isem0, isem1)

        zs = s * rows_per_tile
        pltpu.sync_copy(znd_hbm.at[pl.ds(zs, rows_per_tile)],
                        num_acc.at[pl.ds(zs, rows_per_tile)])
        pltpu.sync_copy(zn_hbm.at[pl.ds(zs, rows_per_tile)],
                        den_acc.at[pl.ds(zs, rows_per_tile)])
        pltpu.sync_copy(as_hbm, as_t)
        pltpu.sync_copy(ad_hbm, ad_t)
        base0 = wid * t_ch * chunk
        pltpu.sync_copy(src_hbm.at[pl.ds(base0, chunk)], sidx.at[0])
        pltpu.sync_copy(dst_hbm.at[pl.ds(base0, chunk)], didx.at[0])
        pltpu.async_copy(src_hbm.at[pl.ds(base0 + chunk, chunk)], sidx.at[1],
                         isem[1])
        pltpu.async_copy(dst_hbm.at[pl.ds(base0 + chunk, chunk)], didx.at[1],
                         isem[1])
        pltpu.async_copy(h_hbm.at[sidx.at[0]], rows2.at[0], gsem[0])
        plsc.subcore_barrier()

        def process(t, b):
            @pl.when(t >= 1)
            def _():
                pltpu.make_async_copy(znd_hbm.at[pl.ds(0, chunk)],
                                      rows2.at[1 - b], rsem[1 - b]).wait()
                pltpu.make_async_copy(zn_hbm.at[pl.ds(0, chunk)],
                                      wbuf.at[1 - b], dsem[1 - b]).wait()

            @pl.when(t + 1 < t_ch)
            def _():
                pltpu.make_async_copy(src_hbm.at[pl.ds(0, chunk)],
                                      sidx.at[1 - b], isem[1 - b]).wait()
                pltpu.make_async_copy(src_hbm.at[pl.ds(0, chunk)],
                                      didx.at[1 - b], isem[1 - b]).wait()
                pltpu.async_copy(h_hbm.at[sidx.at[1 - b]], rows2.at[1 - b],
                                 gsem[1 - b])

            for j in range(chunk // LN):
                s16 = sidx[b, pl.ds(j * LN, LN)]
                d16 = didx[b, pl.ds(j * LN, LN)]
                dscat[b, pl.ds(j * LN, LN)] = d16
                e = plsc.load_gather(as_t, [s16]) + plsc.load_gather(ad_t, [d16])
                e = jnp.where(e > 0.0, e, 0.2 * e)
                wbuf[b, pl.ds(j * LN, LN)] = jnp.exp(e)

            pltpu.async_copy(wbuf.at[b], den_acc.at[dscat.at[b]], dsem[b],
                             add=True)

            pltpu.make_async_copy(znd_hbm.at[pl.ds(0, chunk)],
                                  rows2.at[b], gsem[b]).wait()

            def rowfn(r, _):
                wr = plsc.load_gather(wbuf.at[b], [jnp.full((LN,), r, jnp.int32)])
                for kk in range(D // LN):
                    rows2[b, r, pl.ds(kk * LN, LN)] = (
                        rows2[b, r, pl.ds(kk * LN, LN)] * wr)
                return 0

            lax.fori_loop(0, chunk, rowfn, 0, unroll=8)
            pltpu.async_copy(rows2.at[b], num_acc.at[dscat.at[b]], rsem[b],
                             add=True)

            @pl.when(t + 2 < t_ch)
            def _():
                base = (wid * t_ch + t + 2) * chunk
                pltpu.async_copy(src_hbm.at[pl.ds(base, chunk)], sidx.at[b],
                                 isem[b])
                pltpu.async_copy(dst_hbm.at[pl.ds(base, chunk)], didx.at[b],
                                 isem[b])

        def pair(t2, _):
            process(t2 * 2, 0)
            process(t2 * 2 + 1, 1)
            return 0

        lax.fori_loop(0, t_ch // 2, pair, 0)

        pltpu.make_async_copy(znd_hbm.at[pl.ds(0, chunk)],
                              rows2.at[1], rsem[1]).wait()
        pltpu.make_async_copy(zn_hbm.at[pl.ds(0, chunk)],
                              wbuf.at[1], dsem[1]).wait()
        plsc.subcore_barrier()

        os_ = s * rows_per_tile
        pltpu.sync_copy(num_acc.at[pl.ds(os_, rows_per_tile)],
                        num_out.at[c, pl.ds(os_, rows_per_tile)])
        pltpu.sync_copy(den_acc.at[pl.ds(os_, rows_per_tile)],
                        den_out.at[c, pl.ds(os_, rows_per_tile)])

    return sc_edge


def kernel(x, edge_index, S, W1, a1s, a1d, b1, W2, a2s, a2d, b2, W3, a3s, a3d, b3):
    N, D = x.shape
    Np = ((N + 2047) // 2048) * 2048
    E = edge_index.shape[1]
    Et = E + N
    chunk = 96
    t_ch = -(-Et // (NW * chunk))
    t_ch += t_ch % 2
    E_pad = NW * chunk * t_ch

    pad = N + (jnp.arange(E_pad - Et, dtype=edge_index.dtype) % (Np - N))
    loop = jnp.arange(N, dtype=edge_index.dtype)
    src = jnp.concatenate([edge_index[0], loop, pad])
    dst = jnp.concatenate([edge_index[1], loop, pad])
    xp = jnp.pad(x, ((0, Np - N), (0, 0)))
    znd = jnp.zeros((Np, D), jnp.float32)
    zn = jnp.zeros((Np,), jnp.float32)

    sc_edge = _make_sc_edge(Np, D, E_pad, chunk, t_ch)

    h, asv, adv = _dense_fwd(xp, W1, a1s, a1d)
    num, den = sc_edge(h, asv.reshape(Np), adv.reshape(Np), src, dst, znd, zn)
    h, asv, adv = _combine_fwd(num, den[:, :, None], b1, W2, a2s, a2d)
    num, den = sc_edge(h, asv.reshape(Np), adv.reshape(Np), src, dst, znd, zn)
    h, asv, adv = _combine_fwd(num, den[:, :, None], b2, W3, a3s, a3d)
    num, den = sc_edge(h, asv.reshape(Np), adv.reshape(Np), src, dst, znd, zn)
    out = _final_proj(num, den[:, :, None], b3, S)
    return out[:N]


# speedup vs baseline: 3.8555x; 1.0124x over previous
import functools

import jax
import jax.numpy as jnp
from jax import lax
from jax.experimental import pallas as pl
from jax.experimental.pallas import tpu as pltpu
from jax.experimental.pallas import tpu_sc as plsc

NC = 2
NS = 16
LN = 16
NW = NC * NS


def _dense_fwd(p, W, att_s, att_d, blk=2048):
    Np, D = p.shape

    def body(p_ref, w_ref, s_ref, d_ref, h_ref, as_ref, ad_ref):
        h = jnp.dot(p_ref[...], w_ref[...], preferred_element_type=jnp.float32)
        h_ref[...] = h
        as_ref[...] = jnp.sum(h * s_ref[...], axis=1)[None, :]
        ad_ref[...] = jnp.sum(h * d_ref[...], axis=1)[None, :]

    return pl.pallas_call(
        body,
        grid=(Np // blk,),
        in_specs=[pl.BlockSpec((blk, D), lambda i: (i, 0)),
                  pl.BlockSpec((D, D), lambda i: (0, 0)),
                  pl.BlockSpec((1, D), lambda i: (0, 0)),
                  pl.BlockSpec((1, D), lambda i: (0, 0))],
        out_specs=[pl.BlockSpec((blk, D), lambda i: (i, 0)),
                   pl.BlockSpec((1, blk), lambda i: (0, i)),
                   pl.BlockSpec((1, blk), lambda i: (0, i))],
        out_shape=[jax.ShapeDtypeStruct((Np, D), jnp.float32),
                   jax.ShapeDtypeStruct((1, Np), jnp.float32),
                   jax.ShapeDtypeStruct((1, Np), jnp.float32)],
    )(p, W, att_s[None, :], att_d[None, :])


def _combine_fwd(num, den, bias, W, att_s, att_d, blk=2048):
    _, Np, D = num.shape

    def body(n_ref, d_ref, b_ref, w_ref, s_ref, dd_ref, h_ref, as_ref, ad_ref):
        pre = (n_ref[0] + n_ref[1]) / (d_ref[0] + d_ref[1] + 1e-16) + b_ref[...]
        pre = jnp.where(pre > 0.0, pre, 0.1 * pre)
        h = jnp.dot(pre, w_ref[...], preferred_element_type=jnp.float32)
        h_ref[...] = h
        as_ref[...] = jnp.sum(h * s_ref[...], axis=1)[None, :]
        ad_ref[...] = jnp.sum(h * dd_ref[...], axis=1)[None, :]

    return pl.pallas_call(
        body,
        grid=(Np // blk,),
        in_specs=[pl.BlockSpec((NC, blk, D), lambda i: (0, i, 0)),
                  pl.BlockSpec((NC, blk, 1), lambda i: (0, i, 0)),
                  pl.BlockSpec((1, D), lambda i: (0, 0)),
                  pl.BlockSpec((D, D), lambda i: (0, 0)),
                  pl.BlockSpec((1, D), lambda i: (0, 0)),
                  pl.BlockSpec((1, D), lambda i: (0, 0))],
        out_specs=[pl.BlockSpec((blk, D), lambda i: (i, 0)),
                   pl.BlockSpec((1, blk), lambda i: (0, i)),
                   pl.BlockSpec((1, blk), lambda i: (0, i))],
        out_shape=[jax.ShapeDtypeStruct((Np, D), jnp.float32),
                   jax.ShapeDtypeStruct((1, Np), jnp.float32),
                   jax.ShapeDtypeStruct((1, Np), jnp.float32)],
    )(num, den, bias[None, :], W, att_s[None, :], att_d[None, :])


def _final_proj(num, den, bias, S, blk=2048):
    _, Np, D = num.shape
    K = S.shape[0]

    def body(n_ref, d_ref, b_ref, s_ref, o_ref):
        pre = (n_ref[0] + n_ref[1]) / (d_ref[0] + d_ref[1] + 1e-16) + b_ref[...]
        o_ref[...] = lax.dot_general(pre, s_ref[...], (((1,), (1,)), ((), ())),
                                     preferred_element_type=jnp.float32)

    return pl.pallas_call(
        body,
        grid=(Np // blk,),
        in_specs=[pl.BlockSpec((NC, blk, D), lambda i: (0, i, 0)),
                  pl.BlockSpec((NC, blk, 1), lambda i: (0, i, 0)),
                  pl.BlockSpec((1, D), lambda i: (0, 0)),
                  pl.BlockSpec((K, D), lambda i: (0, 0))],
        out_specs=pl.BlockSpec((blk, K), lambda i: (i, 0)),
        out_shape=jax.ShapeDtypeStruct((Np, K), jnp.float32),
    )(num, den, bias[None, :], S)


def _make_sc_edge(Np, D, E_pad, chunk, t_ch):
    mesh = plsc.VectorSubcoreMesh(core_axis_name="c", subcore_axis_name="s")
    rows_per_tile = Np // NS

    @functools.partial(
        pl.kernel,
        out_type=(jax.ShapeDtypeStruct((NC, Np, D), jnp.float32),
                  jax.ShapeDtypeStruct((NC, Np), jnp.float32)),
        mesh=mesh,
        compiler_params=pltpu.CompilerParams(needs_layout_passes=False),
        scratch_types=[
            pltpu.VMEM((Np,), jnp.float32),
            pltpu.VMEM((Np,), jnp.float32),
            pltpu.VMEM((2, chunk), jnp.int32),
            pltpu.VMEM((2, chunk), jnp.int32),
            pltpu.VMEM((2, chunk), jnp.int32),
            pltpu.VMEM((2, chunk, D), jnp.float32),
            pltpu.VMEM((2, chunk), jnp.float32),
            pltpu.VMEM_SHARED((Np, D), jnp.float32),
            pltpu.VMEM_SHARED((Np,), jnp.float32),
            pltpu.SemaphoreType.DMA,
            pltpu.SemaphoreType.DMA,
            pltpu.SemaphoreType.DMA,
            pltpu.SemaphoreType.DMA,
            pltpu.SemaphoreType.DMA,
            pltpu.SemaphoreType.DMA,
            pltpu.SemaphoreType.DMA,
            pltpu.SemaphoreType.DMA,
        ],
    )
    def sc_edge(h_hbm, as_hbm, ad_hbm, src_hbm, dst_hbm, znd_hbm, zn_hbm,
                num_out, den_out,
                as_t, ad_t, sidx, didx, dscat, rows2, wbuf, num_acc, den_acc,
                gsem0, gsem1, rsem0, rsem1, dsem0, dsem1, isem0, isem1):
        c = lax.axis_index("c")
        s = lax.axis_index("s")
        wid = s * NC + c
        gsem = (gsem0, gsem1)
        rsem = (rsem0, rsem1)
        dsem = (dsem0, dsem1)
        isem = (isem0, isem1)

        zs = s * rows_per_tile
        pltpu.sync_copy(znd_hbm.at[pl.ds(zs, rows_per_tile)],
                        num_acc.at[pl.ds(zs, rows_per_tile)])
        pltpu.sync_copy(zn_hbm.at[pl.ds(zs, rows_per_tile)],
                        den_acc.at[pl.ds(zs, rows_per_tile)])
        pltpu.sync_copy(as_hbm, as_t)
        pltpu.sync_copy(ad_hbm, ad_t)
        base0 = wid * t_ch * chunk
        pltpu.sync_copy(src_hbm.at[pl.ds(base0, chunk)], sidx.at[0])
        pltpu.sync_copy(dst_hbm.at[pl.ds(base0, chunk)], didx.at[0])
        pltpu.async_copy(src_hbm.at[pl.ds(base0 + chunk, chunk)], sidx.at[1],
                         isem[1])
        pltpu.async_copy(dst_hbm.at[pl.ds(base0 + chunk, chunk)], didx.at[1],
                         isem[1])
        pltpu.async_copy(h_hbm.at[sidx.at[0]], rows2.at[0], gsem[0])
        plsc.subcore_barrier()

        def process(t, b):
            @pl.when(t >= 1)
            def _():
                pltpu.make_async_copy(znd_hbm.at[pl.ds(0, chunk)],
                                      rows2.at[1 - b], rsem[1 - b]).wait()
                pltpu.make_async_copy(zn_hbm.at[pl.ds(0, chunk)],
                                      wbuf.at[1 - b], dsem[1 - b]).wait()

            @pl.when(t + 1 < t_ch)
            def _():
                pltpu.make_async_copy(src_hbm.at[pl.ds(0, chunk)],
                                      sidx.at[1 - b], isem[1 - b]).wait()
                pltpu.make_async_copy(src_hbm.at[pl.ds(0, chunk)],
                                      didx.at[1 - b], isem[1 - b]).wait()
                pltpu.async_copy(h_hbm.at[sidx.at[1 - b]], rows2.at[1 - b],
                                 gsem[1 - b])

            for j in range(chunk // LN):
                s16 = sidx[b, pl.ds(j * LN, LN)]
                d16 = didx[b, pl.ds(j * LN, LN)]
                dscat[b, pl.ds(j * LN, LN)] = d16
                e = plsc.load_gather(as_t, [s16]) + plsc.load_gather(ad_t, [d16])
                e = jnp.where(e > 0.0, e, 0.2 * e)
                wbuf[b, pl.ds(j * LN, LN)] = jnp.exp(e)

            pltpu.async_copy(wbuf.at[b], den_acc.at[dscat.at[b]], dsem[b],
                             add=True)

            pltpu.make_async_copy(znd_hbm.at[pl.ds(0, chunk)],
                                  rows2.at[b], gsem[b]).wait()

            def rowfn(r, _):
                wr = plsc.load_gather(wbuf.at[b], [jnp.full((LN,), r, jnp.int32)])
                for kk in range(D // LN):
                    rows2[b, r, pl.ds(kk * LN, LN)] = (
                        rows2[b, r, pl.ds(kk * LN, LN)] * wr)
                return 0

            lax.fori_loop(0, chunk, rowfn, 0, unroll=8)
            pltpu.async_copy(rows2.at[b], num_acc.at[dscat.at[b]], rsem[b],
                             add=True)

            @pl.when(t + 2 < t_ch)
            def _():
                base = (wid * t_ch + t + 2) * chunk
                pltpu.async_copy(src_hbm.at[pl.ds(base, chunk)], sidx.at[b],
                                 isem[b])
                pltpu.async_copy(dst_hbm.at[pl.ds(base, chunk)], didx.at[b],
                                 isem[b])

        def pair(t2, _):
            process(t2 * 2, 0)
            process(t2 * 2 + 1, 1)
            return 0

        lax.fori_loop(0, t_ch // 2, pair, 0)

        pltpu.make_async_copy(znd_hbm.at[pl.ds(0, chunk)],
                              rows2.at[1], rsem[1]).wait()
        pltpu.make_async_copy(zn_hbm.at[pl.ds(0, chunk)],
                              wbuf.at[1], dsem[1]).wait()
        plsc.subcore_barrier()

        os_ = s * rows_per_tile
        pltpu.sync_copy(num_acc.at[pl.ds(os_, rows_per_tile)],
                        num_out.at[c, pl.ds(os_, rows_per_tile)])
        pltpu.sync_copy(den_acc.at[pl.ds(os_, rows_per_tile)],
                        den_out.at[c, pl.ds(os_, rows_per_tile)])

    return sc_edge


def kernel(x, edge_index, S, W1, a1s, a1d, b1, W2, a2s, a2d, b2, W3, a3s, a3d, b3):
    N, D = x.shape
    Np = ((N + 2047) // 2048) * 2048
    E = edge_index.shape[1]
    Et = E + N
    chunk = 96
    t_ch = -(-Et // (NW * chunk))
    t_ch += t_ch % 2
    E_pad = NW * chunk * t_ch

    pad = N + (jnp.arange(E_pad - Et, dtype=edge_index.dtype) % (Np - N))
    loop = jnp.arange(N, dtype=edge_index.dtype)
    src = jnp.concatenate([edge_index[0], loop, pad])
    dst = jnp.concatenate([edge_index[1], loop, pad])
    xp = jnp.pad(x, ((0, Np - N), (0, 0)))
    znd = jnp.zeros((Np, D), jnp.float32)
    zn = jnp.zeros((Np,), jnp.float32)

    sc_edge = _make_sc_edge(Np, D, E_pad, chunk, t_ch)

    h, asv, adv = _dense_fwd(xp, W1, a1s, a1d)
    num, den = sc_edge(h, asv.reshape(Np), adv.reshape(Np), src, dst, znd, zn)
    h, asv, adv = _combine_fwd(num, den[:, :, None], b1, W2, a2s, a2d)
    num, den = sc_edge(h, asv.reshape(Np), adv.reshape(Np), src, dst, znd, zn)
    h, asv, adv = _combine_fwd(num, den[:, :, None], b2, W3, a3s, a3d)
    num, den = sc_edge(h, asv.reshape(Np), adv.reshape(Np), src, dst, znd, zn)
    out = _final_proj(num, den[:, :, None], b3, S)
    return out[:N]


# speedup vs baseline: 4.6036x; 1.1940x over previous
import functools

import jax
import jax.numpy as jnp
from jax import lax
from jax.experimental import pallas as pl
from jax.experimental.pallas import tpu as pltpu
from jax.experimental.pallas import tpu_sc as plsc

NC = 2
NS = 16
LN = 16
NW = NC * NS


def _dense_fwd(p, W, att_s, att_d, blk=2048):
    Np, D = p.shape

    def body(p_ref, w_ref, s_ref, d_ref, h_ref, as_ref, ad_ref):
        h = jnp.dot(p_ref[...], w_ref[...], preferred_element_type=jnp.float32)
        h_ref[...] = h
        as_ref[...] = jnp.sum(h * s_ref[...], axis=1)[None, :]
        ad_ref[...] = jnp.sum(h * d_ref[...], axis=1)[None, :]

    return pl.pallas_call(
        body,
        grid=(Np // blk,),
        in_specs=[pl.BlockSpec((blk, D), lambda i: (i, 0)),
                  pl.BlockSpec((D, D), lambda i: (0, 0)),
                  pl.BlockSpec((1, D), lambda i: (0, 0)),
                  pl.BlockSpec((1, D), lambda i: (0, 0))],
        out_specs=[pl.BlockSpec((blk, D), lambda i: (i, 0)),
                   pl.BlockSpec((1, blk), lambda i: (0, i)),
                   pl.BlockSpec((1, blk), lambda i: (0, i))],
        out_shape=[jax.ShapeDtypeStruct((Np, D), jnp.float32),
                   jax.ShapeDtypeStruct((1, Np), jnp.float32),
                   jax.ShapeDtypeStruct((1, Np), jnp.float32)],
    )(p, W, att_s[None, :], att_d[None, :])


def _combine_fwd(num, den, bias, W, att_s, att_d, blk=2048):
    _, Np, D = num.shape

    def body(n_ref, d_ref, b_ref, w_ref, s_ref, dd_ref, h_ref, as_ref, ad_ref):
        pre = (n_ref[0] + n_ref[1]) / (d_ref[0] + d_ref[1] + 1e-16) + b_ref[...]
        pre = jnp.where(pre > 0.0, pre, 0.1 * pre)
        h = jnp.dot(pre, w_ref[...], preferred_element_type=jnp.float32)
        h_ref[...] = h
        as_ref[...] = jnp.sum(h * s_ref[...], axis=1)[None, :]
        ad_ref[...] = jnp.sum(h * dd_ref[...], axis=1)[None, :]

    return pl.pallas_call(
        body,
        grid=(Np // blk,),
        in_specs=[pl.BlockSpec((NC, blk, D), lambda i: (0, i, 0)),
                  pl.BlockSpec((NC, blk, 1), lambda i: (0, i, 0)),
                  pl.BlockSpec((1, D), lambda i: (0, 0)),
                  pl.BlockSpec((D, D), lambda i: (0, 0)),
                  pl.BlockSpec((1, D), lambda i: (0, 0)),
                  pl.BlockSpec((1, D), lambda i: (0, 0))],
        out_specs=[pl.BlockSpec((blk, D), lambda i: (i, 0)),
                   pl.BlockSpec((1, blk), lambda i: (0, i)),
                   pl.BlockSpec((1, blk), lambda i: (0, i))],
        out_shape=[jax.ShapeDtypeStruct((Np, D), jnp.float32),
                   jax.ShapeDtypeStruct((1, Np), jnp.float32),
                   jax.ShapeDtypeStruct((1, Np), jnp.float32)],
    )(num, den, bias[None, :], W, att_s[None, :], att_d[None, :])


def _final_proj(num, den, bias, S, blk=2048):
    _, Np, D = num.shape
    K = S.shape[0]

    def body(n_ref, d_ref, b_ref, s_ref, o_ref):
        pre = (n_ref[0] + n_ref[1]) / (d_ref[0] + d_ref[1] + 1e-16) + b_ref[...]
        o_ref[...] = lax.dot_general(pre, s_ref[...], (((1,), (1,)), ((), ())),
                                     preferred_element_type=jnp.float32)

    return pl.pallas_call(
        body,
        grid=(Np // blk,),
        in_specs=[pl.BlockSpec((NC, blk, D), lambda i: (0, i, 0)),
                  pl.BlockSpec((NC, blk, 1), lambda i: (0, i, 0)),
                  pl.BlockSpec((1, D), lambda i: (0, 0)),
                  pl.BlockSpec((K, D), lambda i: (0, 0))],
        out_specs=pl.BlockSpec((blk, K), lambda i: (i, 0)),
        out_shape=jax.ShapeDtypeStruct((Np, K), jnp.float32),
    )(num, den, bias[None, :], S)


def _make_sc_edge(Np, D, E_pad, chunk, t_ch):
    mesh = plsc.VectorSubcoreMesh(core_axis_name="c", subcore_axis_name="s")
    rows_per_tile = Np // NS

    @functools.partial(
        pl.kernel,
        out_type=(jax.ShapeDtypeStruct((NC, Np, D), jnp.float32),
                  jax.ShapeDtypeStruct((NC, Np), jnp.float32)),
        mesh=mesh,
        compiler_params=pltpu.CompilerParams(needs_layout_passes=False),
        scratch_types=[
            pltpu.VMEM((Np,), jnp.float32),
            pltpu.VMEM((Np,), jnp.float32),
            pltpu.VMEM((2, chunk), jnp.int32),
            pltpu.VMEM((2, chunk), jnp.int32),
            pltpu.VMEM((2, chunk), jnp.int32),
            pltpu.VMEM((2, chunk, D), jnp.float32),
            pltpu.VMEM((2, chunk), jnp.float32),
            pltpu.VMEM_SHARED((Np, D), jnp.float32),
            pltpu.VMEM_SHARED((Np,), jnp.float32),
            pltpu.SemaphoreType.DMA,
            pltpu.SemaphoreType.DMA,
            pltpu.SemaphoreType.DMA,
            pltpu.SemaphoreType.DMA,
            pltpu.SemaphoreType.DMA,
            pltpu.SemaphoreType.DMA,
            pltpu.SemaphoreType.DMA,
            pltpu.SemaphoreType.DMA,
        ],
    )
    def sc_edge(h_hbm, as_hbm, ad_hbm, src_hbm, dst_hbm, znd_hbm, zn_hbm,
                num_out, den_out,
                as_t, ad_t, sidx, didx, dscat, rows2, wbuf, num_acc, den_acc,
                gsem0, gsem1, rsem0, rsem1, dsem0, dsem1, isem0, isem1):
        c = lax.axis_index("c")
        s = lax.axis_index("s")
        wid = s * NC + c
        gsem = (gsem0, gsem1)
        rsem = (rsem0, rsem1)
        dsem = (dsem0, dsem1)
        isem = (isem0, isem1)

        zs = s * rows_per_tile
        pltpu.sync_copy(znd_hbm.at[pl.ds(zs, rows_per_tile)],
                        num_acc.at[pl.ds(zs, rows_per_tile)])
        pltpu.sync_copy(zn_hbm.at[pl.ds(zs, rows_per_tile)],
                        den_acc.at[pl.ds(zs, rows_per_tile)])
        pltpu.sync_copy(as_hbm, as_t)
        pltpu.sync_copy(ad_hbm, ad_t)
        base0 = wid * t_ch * chunk
        pltpu.sync_copy(src_hbm.at[pl.ds(base0, chunk)], sidx.at[0])
        pltpu.sync_copy(dst_hbm.at[pl.ds(base0, chunk)], didx.at[0])
        pltpu.async_copy(src_hbm.at[pl.ds(base0 + chunk, chunk)], sidx.at[1],
                         isem[1])
        pltpu.async_copy(dst_hbm.at[pl.ds(base0 + chunk, chunk)], didx.at[1],
                         isem[1])
        pltpu.async_copy(h_hbm.at[sidx.at[0]], rows2.at[0], gsem[0])
        plsc.subcore_barrier()

        def process(t, b):
            @pl.when(t >= 1)
            def _():
                pltpu.make_async_copy(znd_hbm.at[pl.ds(0, chunk)],
                                      rows2.at[1 - b], rsem[1 - b]).wait()
                pltpu.make_async_copy(zn_hbm.at[pl.ds(0, chunk)],
                                      wbuf.at[1 - b], dsem[1 - b]).wait()

            @pl.when(t + 1 < t_ch)
            def _():
                pltpu.make_async_copy(src_hbm.at[pl.ds(0, chunk)],
                                      sidx.at[1 - b], isem[1 - b]).wait()
                pltpu.make_async_copy(src_hbm.at[pl.ds(0, chunk)],
                                      didx.at[1 - b], isem[1 - b]).wait()
                pltpu.async_copy(h_hbm.at[sidx.at[1 - b]], rows2.at[1 - b],
                                 gsem[1 - b])

            for j in range(chunk // LN):
                s16 = sidx[b, pl.ds(j * LN, LN)]
                d16 = didx[b, pl.ds(j * LN, LN)]
                dscat[b, pl.ds(j * LN, LN)] = d16
                e = plsc.load_gather(as_t, [s16]) + plsc.load_gather(ad_t, [d16])
                e = jnp.where(e > 0.0, e, 0.2 * e)
                wbuf[b, pl.ds(j * LN, LN)] = jnp.exp(e)

            pltpu.async_copy(wbuf.at[b], den_acc.at[dscat.at[b]], dsem[b],
                             add=True)

            pltpu.make_async_copy(znd_hbm.at[pl.ds(0, chunk)],
                                  rows2.at[b], gsem[b]).wait()

            def grpfn(g, _):
                w16 = wbuf[b, pl.ds(g * LN, LN)]
                for ri in range(LN):
                    wr = jnp.take(w16, jnp.full((LN,), ri, jnp.int32))
                    r = g * LN + ri
                    for kk in range(D // LN):
                        rows2[b, r, pl.ds(kk * LN, LN)] = (
                            rows2[b, r, pl.ds(kk * LN, LN)] * wr)
                return 0

            lax.fori_loop(0, chunk // LN, grpfn, 0)
            pltpu.async_copy(rows2.at[b], num_acc.at[dscat.at[b]], rsem[b],
                             add=True)

            @pl.when(t + 2 < t_ch)
            def _():
                base = (wid * t_ch + t + 2) * chunk
                pltpu.async_copy(src_hbm.at[pl.ds(base, chunk)], sidx.at[b],
                                 isem[b])
                pltpu.async_copy(dst_hbm.at[pl.ds(base, chunk)], didx.at[b],
                                 isem[b])

        def pair(t2, _):
            process(t2 * 2, 0)
            process(t2 * 2 + 1, 1)
            return 0

        lax.fori_loop(0, t_ch // 2, pair, 0)

        pltpu.make_async_copy(znd_hbm.at[pl.ds(0, chunk)],
                              rows2.at[1], rsem[1]).wait()
        pltpu.make_async_copy(zn_hbm.at[pl.ds(0, chunk)],
                              wbuf.at[1], dsem[1]).wait()
        plsc.subcore_barrier()

        os_ = s * rows_per_tile
        pltpu.sync_copy(num_acc.at[pl.ds(os_, rows_per_tile)],
                        num_out.at[c, pl.ds(os_, rows_per_tile)])
        pltpu.sync_copy(den_acc.at[pl.ds(os_, rows_per_tile)],
                        den_out.at[c, pl.ds(os_, rows_per_tile)])

    return sc_edge


def kernel(x, edge_index, S, W1, a1s, a1d, b1, W2, a2s, a2d, b2, W3, a3s, a3d, b3):
    N, D = x.shape
    Np = ((N + 2047) // 2048) * 2048
    E = edge_index.shape[1]
    Et = E + N
    chunk = 96
    t_ch = -(-Et // (NW * chunk))
    t_ch += t_ch % 2
    E_pad = NW * chunk * t_ch

    pad = N + (jnp.arange(E_pad - Et, dtype=edge_index.dtype) % (Np - N))
    loop = jnp.arange(N, dtype=edge_index.dtype)
    src = jnp.concatenate([edge_index[0], loop, pad])
    dst = jnp.concatenate([edge_index[1], loop, pad])
    xp = jnp.pad(x, ((0, Np - N), (0, 0)))
    znd = jnp.zeros((Np, D), jnp.float32)
    zn = jnp.zeros((Np,), jnp.float32)

    sc_edge = _make_sc_edge(Np, D, E_pad, chunk, t_ch)

    h, asv, adv = _dense_fwd(xp, W1, a1s, a1d)
    num, den = sc_edge(h, asv.reshape(Np), adv.reshape(Np), src, dst, znd, zn)
    h, asv, adv = _combine_fwd(num, den[:, :, None], b1, W2, a2s, a2d)
    num, den = sc_edge(h, asv.reshape(Np), adv.reshape(Np), src, dst, znd, zn)
    h, asv, adv = _combine_fwd(num, den[:, :, None], b2, W3, a3s, a3d)
    num, den = sc_edge(h, asv.reshape(Np), adv.reshape(Np), src, dst, znd, zn)
    out = _final_proj(num, den[:, :, None], b3, S)
    return out[:N]
